# baseline (speedup 1.0000x reference)
import functools

import jax
import jax.numpy as jnp
from jax import lax
from jax.experimental import pallas as pl
from jax.experimental.pallas import tpu as pltpu
from jax.experimental.pallas import tpu_sc as plsc

_ZCH = 1_000_000
_L = 16
_NC = 2
_NS = 16
_NW = _NC * _NS
_RUN = 128
_CPT = 244 * _RUN
_MAIN = _CPT * _NW
_NEXTRA = 4
_TAIL = _MAIN + _NEXTRA * _RUN


def _count_body(idT_hbm, out_hbm, buf, ebuf, tbuf, accbuf, sem):
    w = lax.axis_index("s") * _NC + lax.axis_index("c")
    accbuf[...] = jnp.zeros((_L,), jnp.int32)
    pltpu.sync_copy(accbuf, out_hbm.at[w])


_count_partials = functools.partial(
    pl.kernel,
    out_type=jax.ShapeDtypeStruct((_NW, _L), jnp.int32),
    mesh=plsc.VectorSubcoreMesh(core_axis_name="c", subcore_axis_name="s"),
    scratch_types=[
        pltpu.VMEM((_CPT,), jnp.int32),
        pltpu.VMEM((_RUN,), jnp.int32),
        pltpu.VMEM((64,), jnp.int32),
        pltpu.VMEM((_L,), jnp.int32),
        pltpu.SemaphoreType.DMA,
    ],
)(_count_body)


def _finish_body(p_ref, cnt_ref, ratio_ref):
    total = jnp.sum(p_ref[...])
    cnt_ref[0, 0] = total
    ratio_ref[0, 0] = (
        jnp.float32(_ZCH) - total.astype(jnp.float32)
    ) / jnp.float32(_ZCH)


def kernel(identities):
    partials = _count_partials(identities.T)
    total = jnp.sum(partials)
    ratio = (jnp.float32(_ZCH) - total.astype(jnp.float32)) / jnp.float32(_ZCH)
    return total.astype(jnp.int32), ratio


# speedup vs baseline: 1.1218x; 1.1218x over previous
import jax
import jax.numpy as jnp
from jax import lax
from jax.experimental import pallas as pl
from jax.experimental.pallas import tpu as pltpu

_ZCH = 1_000_000
_B = 62464
_G = (_ZCH + _B - 1) // _B


def _body(x_ref, cnt_ref, ratio_ref):
    i = pl.program_id(0)
    x = x_ref[...]
    col = lax.broadcasted_iota(jnp.int32, x.shape, 1) + i * _B
    row = lax.broadcasted_iota(jnp.int32, x.shape, 0)
    cnt = jnp.sum(
        jnp.where((x == -1) & (row == 0) & (col < _ZCH), 1, 0).astype(jnp.int32)
    )

    @pl.when(i == 0)
    def _():
        cnt_ref[0, 0] = 0

    cnt_ref[0, 0] += cnt

    @pl.when(i == _G - 1)
    def _():
        total = cnt_ref[0, 0]
        ratio_ref[0, 0] = (
            jnp.float32(_ZCH) - total.astype(jnp.float32)
        ) / jnp.float32(_ZCH)


def kernel(identities):
    idT = identities.T
    cnt, ratio = pl.pallas_call(
        _body,
        grid=(_G,),
        in_specs=[pl.BlockSpec((2, _B), lambda i: (0, i))],
        out_specs=(
            pl.BlockSpec(memory_space=pltpu.SMEM),
            pl.BlockSpec(memory_space=pltpu.SMEM),
        ),
        out_shape=(
            jax.ShapeDtypeStruct((1, 1), jnp.int32),
            jax.ShapeDtypeStruct((1, 1), jnp.float32),
        ),
    )(idT)
    return cnt[0, 0], ratio[0, 0]


# speedup vs baseline: 3.5983x; 3.2075x over previous
import jax
import jax.numpy as jnp
from jax import lax
from jax.experimental import pallas as pl
from jax.experimental.pallas import tpu as pltpu

_ZCH = 1_000_000
_B = 124928
_NSUB = 8
_B8 = _B // _NSUB
_NCH = 8
_TAILC = _NCH * _B
_TAIL = _ZCH - _TAILC


def _body(x_hbm, cnt_ref, ratio_ref, buf, tbuf, s0, s1, s2, ts):
    sems = [s0, s1, s2]

    def cps(k):
        base = k * _B
        return [
            pltpu.make_async_copy(
                x_hbm.at[0, pl.ds(base + j * _B8, _B8)],
                buf.at[k % 3, j],
                sems[k % 3],
            )
            for j in range(_NSUB)
        ]

    def start(k):
        for c in cps(k):
            c.start()

    def wait(k):
        for c in cps(k):
            c.wait()

    start(0)
    start(1)
    tc = pltpu.make_async_copy(x_hbm.at[0, pl.ds(_TAILC, _TAIL)], tbuf, ts)
    tc.start()

    total = jnp.int32(0)
    for k in range(_NCH):
        wait(k)
        if k + 2 < _NCH:
            start(k + 2)
        x = buf[k % 3]
        total = total + jnp.sum((x == -1).astype(jnp.int32))

    tc.wait()
    total = total + jnp.sum((tbuf[...] == -1).astype(jnp.int32))

    cnt_ref[0, 0] = total
    ratio_ref[0, 0] = (
        jnp.float32(_ZCH) - total.astype(jnp.float32)
    ) / jnp.float32(_ZCH)


def kernel(identities):
    idT = identities.T
    cnt, ratio = pl.pallas_call(
        _body,
        compiler_params=pltpu.CompilerParams(
            vmem_limit_bytes=56 * 1024 * 1024
        ),
        in_specs=[pl.BlockSpec(memory_space=pl.ANY)],
        out_specs=(
            pl.BlockSpec(memory_space=pltpu.SMEM),
            pl.BlockSpec(memory_space=pltpu.SMEM),
        ),
        out_shape=(
            jax.ShapeDtypeStruct((1, 1), jnp.int32),
            jax.ShapeDtypeStruct((1, 1), jnp.float32),
        ),
        scratch_shapes=[
            pltpu.VMEM((3, _NSUB, _B8), jnp.int32),
            pltpu.VMEM((_TAIL,), jnp.int32),
            pltpu.SemaphoreType.DMA,
            pltpu.SemaphoreType.DMA,
            pltpu.SemaphoreType.DMA,
            pltpu.SemaphoreType.DMA,
        ],
    )(idT)
    return cnt[0, 0], ratio[0, 0]


# speedup vs baseline: 4.9087x; 1.3642x over previous
import jax
import jax.numpy as jnp
from jax import lax
from jax.experimental import pallas as pl
from jax.experimental.pallas import tpu as pltpu

_ZCH = 1_000_000
_B = 124928
_NSUB = 8
_B8 = _B // _NSUB
_NCH = 8
_TAILC = _NCH * _B
_TAIL = _ZCH - _TAILC


def _body(x_hbm, cnt_ref, ratio_ref, buf, tbuf, s0, s1, s2, s3, s4, s5, s6, s7, ts):
    sems = [s0, s1, s2, s3, s4, s5, s6, s7]

    def cps(k):
        base = k * _B
        return [
            pltpu.make_async_copy(
                x_hbm.at[0, pl.ds(base + j * _B8, _B8)],
                buf.at[k, j],
                sems[k],
            )
            for j in range(_NSUB)
        ]

    for k in range(_NCH):
        for c in cps(k):
            c.start()
    tc = pltpu.make_async_copy(x_hbm.at[0, pl.ds(_TAILC, _TAIL)], tbuf, ts)
    tc.start()

    total = jnp.int32(0)
    for k in range(_NCH):
        for c in cps(k):
            c.wait()
        total = total + jnp.sum((buf[k] == -1).astype(jnp.int32))

    tc.wait()
    total = total + jnp.sum((tbuf[...] == -1).astype(jnp.int32))

    cnt_ref[0, 0] = total
    ratio_ref[0, 0] = (
        jnp.float32(_ZCH) - total.astype(jnp.float32)
    ) / jnp.float32(_ZCH)


def kernel(identities):
    idT = identities.T
    cnt, ratio = pl.pallas_call(
        _body,
        compiler_params=pltpu.CompilerParams(
            vmem_limit_bytes=56 * 1024 * 1024
        ),
        in_specs=[pl.BlockSpec(memory_space=pl.ANY)],
        out_specs=(
            pl.BlockSpec(memory_space=pltpu.SMEM),
            pl.BlockSpec(memory_space=pltpu.SMEM),
        ),
        out_shape=(
            jax.ShapeDtypeStruct((1, 1), jnp.int32),
            jax.ShapeDtypeStruct((1, 1), jnp.float32),
        ),
        scratch_shapes=[
            pltpu.VMEM((_NCH, _NSUB, _B8), jnp.int32),
            pltpu.VMEM((_TAIL,), jnp.int32),
        ] + [pltpu.SemaphoreType.DMA] * 9,
    )(idT)
    return cnt[0, 0], ratio[0, 0]


# speedup vs baseline: 5.4402x; 1.1083x over previous
import jax
import jax.numpy as jnp
from jax import lax
from jax.experimental import pallas as pl
from jax.experimental.pallas import tpu as pltpu

_ZCH = 1_000_000
_B = 62464
_NSUB = 8
_B8 = _B // _NSUB
_NCH = 16
_TAILC = _NCH * _B
_TAIL = _ZCH - _TAILC


def _body(x_hbm, cnt_ref, ratio_ref, buf, tbuf, *sems_all):
    sems = list(sems_all[:-1])
    ts = sems_all[-1]

    def cps(k):
        base = k * _B
        return [
            pltpu.make_async_copy(
                x_hbm.at[0, pl.ds(base + j * _B8, _B8)],
                buf.at[k, j],
                sems[k],
            )
            for j in range(_NSUB)
        ]

    for k in range(_NCH):
        for c in cps(k):
            c.start()
    tc = pltpu.make_async_copy(x_hbm.at[0, pl.ds(_TAILC, _TAIL)], tbuf, ts)
    tc.start()

    acc = jnp.zeros((_NSUB, 128), jnp.int32)
    ones = jnp.ones((_NSUB, _B8), jnp.int32)
    zeros = jnp.zeros((_NSUB, _B8), jnp.int32)
    for k in range(_NCH):
        for c in cps(k):
            c.wait()
        hit = jnp.where(buf[k] == -1, ones, zeros)
        acc = acc + jnp.sum(hit.reshape(_NSUB, _B8 // 128, 128), axis=1)

    total = jnp.sum(acc)
    tc.wait()
    total = total + jnp.sum((tbuf[...] == -1).astype(jnp.int32))

    cnt_ref[0, 0] = total
    ratio_ref[0, 0] = (
        jnp.float32(_ZCH) - total.astype(jnp.float32)
    ) / jnp.float32(_ZCH)


def kernel(identities):
    idT = identities.T
    cnt, ratio = pl.pallas_call(
        _body,
        compiler_params=pltpu.CompilerParams(
            vmem_limit_bytes=56 * 1024 * 1024
        ),
        in_specs=[pl.BlockSpec(memory_space=pl.ANY)],
        out_specs=(
            pl.BlockSpec(memory_space=pltpu.SMEM),
            pl.BlockSpec(memory_space=pltpu.SMEM),
        ),
        out_shape=(
            jax.ShapeDtypeStruct((1, 1), jnp.int32),
            jax.ShapeDtypeStruct((1, 1), jnp.float32),
        ),
        scratch_shapes=[
            pltpu.VMEM((_NCH, _NSUB, _B8), jnp.int32),
            pltpu.VMEM((_TAIL,), jnp.int32),
        ] + [pltpu.SemaphoreType.DMA] * (_NCH + 1),
    )(idT)
    return cnt[0, 0], ratio[0, 0]
